# baseline (speedup 1.0000x reference)
import functools

import jax
import jax.numpy as jnp
import numpy as np
from jax import lax
from jax.experimental import pallas as pl
from jax.experimental.pallas import tpu as pltpu
from jax.experimental.pallas import tpu_sc as plsc

N = 10000
E = 320000
IN_FEATS = 128
H_FEATS = 32
PROP_STEP = 8

NP = 10240
F = 16
NSUB = 16
RPN = NP // NSUB
CR = 128
NSC = RPN // CR

IB = 128
EP_PAD = 327680
ROWS_G = 2 * EP_PAD // IB
ROWS_T = ROWS_G // NSUB
RB = 40
NBUF = 8
HALF = NBUF // 2
NCHUNK = ROWS_T // RB
DROWS_T = (EP_PAD // IB) // NSUB
DCHUNK = DROWS_T // RB
ZR = 32


def _rsqrt16(d):
    y = jnp.ones((16,), jnp.float32)
    for _ in range(7):
        y = jnp.where(d * y * y > 1.5, y * 0.5, y)
    for _ in range(5):
        y = y * (1.5 - 0.5 * d * y * y)
    return y


def _mlp_body(x_ref, w1_ref, a_ref, c_ref, w2_ref, b2_ref, o_ref):
    h = jnp.dot(x_ref[...], w1_ref[...], preferred_element_type=jnp.float32)
    h = jnp.maximum(h * a_ref[...] + c_ref[...], 0.0)
    o_ref[...] = jnp.dot(h, w2_ref[...], preferred_element_type=jnp.float32) + b2_ref[...]


def _sc_body(src2d, dst2d, ori, out,
             hp, hnh, aggp, aggn, degop, degip, degon, degin,
             src_a, dst_a, src_b, dst_b,
             rb0, rb1, rb2, rb3, rb4, rb5, rb6, rb7, obuf, zrow,
             o_c, ap_c, an_c, h_c, sipb, sinb, sopb, sonb, stab,
             gs0, gs1, gs2, gs3, gs4, gs5, gs6, gs7,
             ss0, ss1, ss2, ss3, ss4, ss5, ss6, ss7, is0, is1):
    src_v, dst_v = src_a, dst_a
    cid0 = lax.axis_index("c")
    hn = hnh.at[cid0]
    rbufs = (rb0, rb1, rb2, rb3, rb4, rb5, rb6, rb7)
    gsems = (gs0, gs1, gs2, gs3, gs4, gs5, gs6, gs7)
    ssems = (ss0, ss1, ss2, ss3, ss4, ss5, ss6, ss7)
    cid = lax.axis_index("c")
    sid = lax.axis_index("s")
    r0 = sid * RPN
    ones = jnp.ones((16,), jnp.float32)
    zvec = jnp.zeros((16,), jnp.float32)

    def fill_zrow(i, _):
        zrow[i, :] = zvec
        return 0
    lax.fori_loop(0, ZR, fill_zrow, 0)

    def fill_ones(i, _):
        obuf[pl.ds(i * 16, 16)] = ones
        stab[pl.ds(i * 16, 16)] = zvec
        return 0
    lax.fori_loop(0, IB // 16, fill_ones, 0)

    def fill_stab(i, _):
        stab[pl.ds(IB + i * 16, 16)] = zvec
        return 0
    lax.fori_loop(0, (RPN - IB) // 16, fill_stab, 0)

    for tbl in (aggp, aggn):
        for j in range(RPN // ZR):
            pltpu.sync_copy(zrow, tbl.at[pl.ds(r0 + j * ZR, ZR)])
    for tbl in (degop, degip, degon, degin):
        pltpu.sync_copy(stab, tbl.at[pl.ds(r0, RPN)])
    plsc.subcore_barrier()

    for tbl, arr2d, base in ((degop, src2d, 0), (degip, dst2d, 0),
                             (degon, src2d, EP_PAD // IB),
                             (degin, dst2d, EP_PAD // IB)):
        def dchunk(c, _, tbl=tbl, arr2d=arr2d, base=base):
            row0 = base + sid * DROWS_T + c * RB
            pltpu.sync_copy(arr2d.at[pl.ds(row0, RB)], src_v)

            sdescs = [None] * NBUF
            for r in range(RB):
                b = r % NBUF
                if sdescs[b] is not None:
                    sdescs[b].wait()
                sdescs[b] = pltpu.async_copy(obuf, tbl.at[src_v.at[r]],
                                             ssems[b], add=True)
            for b in range(NBUF):
                if sdescs[b] is not None:
                    sdescs[b].wait()
            return 0
        lax.fori_loop(0, DCHUNK, dchunk, 0)
    plsc.subcore_barrier()

    for tbl, sb in ((degop, sopb), (degip, sipb), (degon, sonb), (degin, sinb)):
        pltpu.sync_copy(tbl.at[pl.ds(r0, RPN)], stab)

        def srow(i, _):
            d = stab[pl.ds(i * 16, 16)]
            stab[pl.ds(i * 16, 16)] = _rsqrt16(jnp.maximum(d, 1.0))
            return 0
        lax.fori_loop(0, RPN // 16, srow, 0)

        def sexp(i, _, sb=sb):
            sb[i, :] = plsc.load_gather(stab, [jnp.full((16,), i, jnp.int32)])
            return 0
        lax.fori_loop(0, RPN, sexp, 0)

    for j in range(NSC):
        rj = r0 + j * CR
        pltpu.sync_copy(ori.at[cid, pl.ds(rj, CR)], o_c)

        def irow(i, _, j=j):
            hrow = o_c[i, :]
            ap_c[i, :] = hrow * sopb[j * CR + i, :]
            an_c[i, :] = hrow * sonb[j * CR + i, :]
            return 0
        lax.fori_loop(0, CR, irow, 0)
        pltpu.sync_copy(ap_c, hp.at[pl.ds(rj, CR)])
        pltpu.sync_copy(an_c, hn.at[pl.ds(rj, CR)])
    plsc.subcore_barrier()

    def step(_, carry):
        def ring(sv, dv, htab, atab):
            gdescs = [None] * NBUF
            sdescs = [None] * NBUF
            for b in range(HALF):
                gdescs[b] = pltpu.async_copy(htab.at[sv.at[b]], rbufs[b],
                                             gsems[b])
            for r in range(RB):
                b = r % NBUF
                gdescs[b].wait()
                sdescs[b] = pltpu.async_copy(rbufs[b], atab.at[dv.at[r]],
                                             ssems[b], add=True)
                rg = r + HALF
                if rg < RB:
                    b2 = rg % NBUF
                    if sdescs[b2] is not None:
                        sdescs[b2].wait()
                        sdescs[b2] = None
                    gdescs[b2] = pltpu.async_copy(htab.at[sv.at[rg]],
                                                  rbufs[b2], gsems[b2])
            for b in range(NBUF):
                if sdescs[b] is not None:
                    sdescs[b].wait()

        def edge_chunk(c, __, htab=None, atab=None):
            row1 = sid * ROWS_T + (c + 1) * RB

            def one_parity(cs, cd, ns, nd):
                @pl.when(c < NCHUNK - 1)
                def _():
                    pltpu.async_copy(src2d.at[pl.ds(row1, RB)], ns, is0)
                    pltpu.async_copy(dst2d.at[pl.ds(row1, RB)], nd, is1)
                ring(cs, cd, htab, atab)
                @pl.when(c < NCHUNK - 1)
                def _():
                    pltpu.make_async_copy(src2d.at[pl.ds(row1, RB)], ns,
                                          is0).wait()
                    pltpu.make_async_copy(dst2d.at[pl.ds(row1, RB)], nd,
                                          is1).wait()

            @pl.when(c % 2 == 0)
            def _():
                one_parity(src_a, dst_a, src_b, dst_b)

            @pl.when(c % 2 == 1)
            def _():
                one_parity(src_b, dst_b, src_a, dst_a)
            return 0

        row00 = sid * ROWS_T
        pltpu.sync_copy(src2d.at[pl.ds(row00, RB)], src_a)
        pltpu.sync_copy(dst2d.at[pl.ds(row00, RB)], dst_a)

        @pl.when(sid < 8)
        def _():
            lax.fori_loop(0, NCHUNK, functools.partial(edge_chunk, htab=hp, atab=aggp), 0)

        @pl.when(sid >= 8)
        def _():
            lax.fori_loop(0, NCHUNK, functools.partial(edge_chunk, htab=hn, atab=aggn), 0)

        plsc.subcore_barrier()

        for j in range(NSC):
            rj = r0 + j * CR
            pltpu.sync_copy(aggp.at[pl.ds(rj, CR)], ap_c)
            pltpu.sync_copy(aggn.at[pl.ds(rj, CR)], an_c)
            pltpu.sync_copy(ori.at[cid, pl.ds(rj, CR)], o_c)

            def prow(i, __, j=j):
                hrow = (ap_c[i, :] * sipb[j * CR + i, :]
                        - an_c[i, :] * sinb[j * CR + i, :] + o_c[i, :])
                h_c[i, :] = hrow
                ap_c[i, :] = hrow * sopb[j * CR + i, :]
                an_c[i, :] = hrow * sonb[j * CR + i, :]
                return 0
            lax.fori_loop(0, CR, prow, 0)
            pltpu.sync_copy(ap_c, hp.at[pl.ds(rj, CR)])
            pltpu.sync_copy(an_c, hn.at[pl.ds(rj, CR)])
            for z in range(CR // ZR):
                pltpu.sync_copy(zrow, aggp.at[pl.ds(rj + z * ZR, ZR)])
                pltpu.sync_copy(zrow, aggn.at[pl.ds(rj + z * ZR, ZR)])
            pltpu.sync_copy(h_c, out.at[cid, pl.ds(rj, CR)])
        plsc.subcore_barrier()
        return carry

    lax.fori_loop(0, PROP_STEP, step, 0)


_sc_call = pl.kernel(
    _sc_body,
    out_type=jax.ShapeDtypeStruct((2, NP, F), jnp.float32),
    mesh=plsc.VectorSubcoreMesh(core_axis_name="c", subcore_axis_name="s",
                                num_cores=2, num_subcores=NSUB),
    compiler_params=pltpu.CompilerParams(use_tc_tiling_on_sc=False,
                                         needs_layout_passes=False),
    scratch_types=[
        pltpu.VMEM_SHARED((NP, F), jnp.float32),
        pltpu.HBM((2, NP, F), jnp.float32),
        pltpu.VMEM_SHARED((NP, F), jnp.float32),
        pltpu.VMEM_SHARED((NP, F), jnp.float32),
        pltpu.VMEM_SHARED((NP,), jnp.float32),
        pltpu.VMEM_SHARED((NP,), jnp.float32),
        pltpu.VMEM_SHARED((NP,), jnp.float32),
        pltpu.VMEM_SHARED((NP,), jnp.float32),
        pltpu.VMEM((RB, IB), jnp.int32),
        pltpu.VMEM((RB, IB), jnp.int32),
        pltpu.VMEM((RB, IB), jnp.int32),
        pltpu.VMEM((RB, IB), jnp.int32),
        *([pltpu.VMEM((IB, F), jnp.float32)] * NBUF),
        pltpu.VMEM((IB,), jnp.float32),
        pltpu.VMEM((ZR, F), jnp.float32),
        pltpu.VMEM((CR, F), jnp.float32),
        pltpu.VMEM((CR, F), jnp.float32),
        pltpu.VMEM((CR, F), jnp.float32),
        pltpu.VMEM((CR, F), jnp.float32),
        pltpu.VMEM((RPN, F), jnp.float32),
        pltpu.VMEM((RPN, F), jnp.float32),
        pltpu.VMEM((RPN, F), jnp.float32),
        pltpu.VMEM((RPN, F), jnp.float32),
        pltpu.VMEM((RPN,), jnp.float32),
        *([pltpu.SemaphoreType.DMA] * (2 * NBUF + 2)),
    ],
)


def kernel(x, edge_index, neg_edge_index, W1, b1, gamma, beta, W2, b2):
    scale = (gamma * np.float32(1.0 / np.sqrt(1.0 + 1e-5))).reshape(1, H_FEATS)
    shift = (b1 * scale[0] + beta).reshape(1, H_FEATS)
    ori = pl.pallas_call(
        _mlp_body,
        out_shape=jax.ShapeDtypeStruct((N, H_FEATS), jnp.float32),
    )(x, W1, scale, shift, W2, b2.reshape(1, H_FEATS))

    ori_split = jnp.pad(ori, ((0, NP - N), (0, 0))).reshape(NP, 2, F).transpose(1, 0, 2)

    padv = jnp.full((EP_PAD - E,), NP - 1, dtype=jnp.int32)
    src2d = jnp.concatenate(
        [edge_index[0], padv, neg_edge_index[0], padv]).reshape(ROWS_G, IB)
    dst2d = jnp.concatenate(
        [edge_index[1], padv, neg_edge_index[1], padv]).reshape(ROWS_G, IB)

    out = _sc_call(src2d, dst2d, ori_split)
    return out.transpose(1, 0, 2).reshape(NP, H_FEATS)[:N]


# speedup vs baseline: 1.3794x; 1.3794x over previous
import functools

import jax
import jax.numpy as jnp
import numpy as np
from jax import lax
from jax.experimental import pallas as pl
from jax.experimental.pallas import tpu as pltpu
from jax.experimental.pallas import tpu_sc as plsc

N = 10000
E = 320000
IN_FEATS = 128
H_FEATS = 32
PROP_STEP = 8

NP = 10240
F = 16
NSUB = 16
RPN = NP // NSUB
CR = 128
NSC = RPN // CR

IB = 128
EP_PAD = 327680
ROWS_G = 2 * EP_PAD // IB
ROWS_T = ROWS_G // NSUB
RB = 40
NBUF = 8
HALF = NBUF // 2
NCHUNK = ROWS_T // RB
DROWS_T = (EP_PAD // IB) // NSUB
DCHUNK = DROWS_T // RB
ZR = 32


def _rsqrt16(d):
    y = jnp.ones((16,), jnp.float32)
    for _ in range(7):
        y = jnp.where(d * y * y > 1.5, y * 0.5, y)
    for _ in range(5):
        y = y * (1.5 - 0.5 * d * y * y)
    return y


def _mlp_body(x_ref, w1_ref, a_ref, c_ref, w2_ref, b2_ref, o_ref):
    h = jnp.dot(x_ref[...], w1_ref[...], preferred_element_type=jnp.float32)
    h = jnp.maximum(h * a_ref[...] + c_ref[...], 0.0)
    o_ref[...] = jnp.dot(h, w2_ref[...], preferred_element_type=jnp.float32) + b2_ref[...]


def _sc_body(src2d, dst2d, ori, out,
             hp, hn, aggp, aggn, degop, degip, degon, degin,
             src_a, dst_a, src_b, dst_b,
             rb0, rb1, rb2, rb3, rb4, rb5, rb6, rb7, obuf, zrow,
             o_c, ap_c, an_c, h_c, sipb, sinb, sopb, sonb, stab,
             gs0, gs1, gs2, gs3, gs4, gs5, gs6, gs7,
             ss0, ss1, ss2, ss3, ss4, ss5, ss6, ss7, is0, is1):
    src_v, dst_v = src_a, dst_a
    rbufs = (rb0, rb1, rb2, rb3, rb4, rb5, rb6, rb7)
    gsems = (gs0, gs1, gs2, gs3, gs4, gs5, gs6, gs7)
    ssems = (ss0, ss1, ss2, ss3, ss4, ss5, ss6, ss7)
    cid = lax.axis_index("c")
    sid = lax.axis_index("s")
    r0 = sid * RPN
    ones = jnp.ones((16,), jnp.float32)
    zvec = jnp.zeros((16,), jnp.float32)

    def fill_zrow(i, _):
        zrow[i, :] = zvec
        return 0
    lax.fori_loop(0, ZR, fill_zrow, 0)

    def fill_ones(i, _):
        obuf[pl.ds(i * 16, 16)] = ones
        stab[pl.ds(i * 16, 16)] = zvec
        return 0
    lax.fori_loop(0, IB // 16, fill_ones, 0)

    def fill_stab(i, _):
        stab[pl.ds(IB + i * 16, 16)] = zvec
        return 0
    lax.fori_loop(0, (RPN - IB) // 16, fill_stab, 0)

    for tbl in (aggp, aggn):
        for j in range(RPN // ZR):
            pltpu.sync_copy(zrow, tbl.at[pl.ds(r0 + j * ZR, ZR)])
    for tbl in (degop, degip, degon, degin):
        pltpu.sync_copy(stab, tbl.at[pl.ds(r0, RPN)])
    plsc.subcore_barrier()

    for tbl, arr2d, base in ((degop, src2d, 0), (degip, dst2d, 0),
                             (degon, src2d, EP_PAD // IB),
                             (degin, dst2d, EP_PAD // IB)):
        def dchunk(c, _, tbl=tbl, arr2d=arr2d, base=base):
            row0 = base + sid * DROWS_T + c * RB
            pltpu.sync_copy(arr2d.at[pl.ds(row0, RB)], src_v)

            sdescs = [None] * NBUF
            for r in range(RB):
                b = r % NBUF
                if sdescs[b] is not None:
                    sdescs[b].wait()
                sdescs[b] = pltpu.async_copy(obuf, tbl.at[src_v.at[r]],
                                             ssems[b], add=True)
            for b in range(NBUF):
                if sdescs[b] is not None:
                    sdescs[b].wait()
            return 0
        lax.fori_loop(0, DCHUNK, dchunk, 0)
    plsc.subcore_barrier()

    for tbl, sb in ((degop, sopb), (degip, sipb), (degon, sonb), (degin, sinb)):
        pltpu.sync_copy(tbl.at[pl.ds(r0, RPN)], stab)

        def srow(i, _):
            d = stab[pl.ds(i * 16, 16)]
            stab[pl.ds(i * 16, 16)] = _rsqrt16(jnp.maximum(d, 1.0))
            return 0
        lax.fori_loop(0, RPN // 16, srow, 0)

        def sexp(i, _, sb=sb):
            sb[i, :] = plsc.load_gather(stab, [jnp.full((16,), i, jnp.int32)])
            return 0
        lax.fori_loop(0, RPN, sexp, 0)

    for j in range(NSC):
        rj = r0 + j * CR
        pltpu.sync_copy(ori.at[cid, pl.ds(rj, CR)], o_c)

        def irow(i, _, j=j):
            hrow = o_c[i, :]
            ap_c[i, :] = hrow * sopb[j * CR + i, :]
            an_c[i, :] = hrow * sonb[j * CR + i, :]
            return 0
        lax.fori_loop(0, CR, irow, 0)
        pltpu.sync_copy(ap_c, hp.at[pl.ds(rj, CR)])
        pltpu.sync_copy(an_c, hn.at[pl.ds(rj, CR)])
    plsc.subcore_barrier()

    def step(_, carry):
        def ring(sv, dv, htab, atab):
            gdescs = [None] * NBUF
            sdescs = [None] * NBUF
            for b in range(HALF):
                gdescs[b] = pltpu.async_copy(htab.at[sv.at[b]], rbufs[b],
                                             gsems[b])
            for r in range(RB):
                b = r % NBUF
                gdescs[b].wait()
                sdescs[b] = pltpu.async_copy(rbufs[b], atab.at[dv.at[r]],
                                             ssems[b], add=True)
                rg = r + HALF
                if rg < RB:
                    b2 = rg % NBUF
                    if sdescs[b2] is not None:
                        sdescs[b2].wait()
                        sdescs[b2] = None
                    gdescs[b2] = pltpu.async_copy(htab.at[sv.at[rg]],
                                                  rbufs[b2], gsems[b2])
            for b in range(NBUF):
                if sdescs[b] is not None:
                    sdescs[b].wait()

        def edge_chunk(c, __, htab=None, atab=None):
            row1 = sid * ROWS_T + (c + 1) * RB

            def one_parity(cs, cd, ns, nd):
                @pl.when(c < NCHUNK - 1)
                def _():
                    pltpu.async_copy(src2d.at[pl.ds(row1, RB)], ns, is0)
                    pltpu.async_copy(dst2d.at[pl.ds(row1, RB)], nd, is1)
                ring(cs, cd, htab, atab)
                @pl.when(c < NCHUNK - 1)
                def _():
                    pltpu.make_async_copy(src2d.at[pl.ds(row1, RB)], ns,
                                          is0).wait()
                    pltpu.make_async_copy(dst2d.at[pl.ds(row1, RB)], nd,
                                          is1).wait()

            @pl.when(c % 2 == 0)
            def _():
                one_parity(src_a, dst_a, src_b, dst_b)

            @pl.when(c % 2 == 1)
            def _():
                one_parity(src_b, dst_b, src_a, dst_a)
            return 0

        row00 = sid * ROWS_T
        pltpu.sync_copy(src2d.at[pl.ds(row00, RB)], src_a)
        pltpu.sync_copy(dst2d.at[pl.ds(row00, RB)], dst_a)

        @pl.when(sid < 8)
        def _():
            lax.fori_loop(0, NCHUNK, functools.partial(edge_chunk, htab=hp, atab=aggp), 0)

        @pl.when(sid >= 8)
        def _():
            lax.fori_loop(0, NCHUNK, functools.partial(edge_chunk, htab=hn, atab=aggn), 0)

        plsc.subcore_barrier()

        for j in range(NSC):
            rj = r0 + j * CR
            pltpu.sync_copy(aggp.at[pl.ds(rj, CR)], ap_c)
            pltpu.sync_copy(aggn.at[pl.ds(rj, CR)], an_c)
            pltpu.sync_copy(ori.at[cid, pl.ds(rj, CR)], o_c)

            def prow(i, __, j=j):
                hrow = (ap_c[i, :] * sipb[j * CR + i, :]
                        - an_c[i, :] * sinb[j * CR + i, :] + o_c[i, :])
                h_c[i, :] = hrow
                ap_c[i, :] = hrow * sopb[j * CR + i, :]
                an_c[i, :] = hrow * sonb[j * CR + i, :]
                return 0
            lax.fori_loop(0, CR, prow, 0)
            pltpu.sync_copy(ap_c, hp.at[pl.ds(rj, CR)])
            pltpu.sync_copy(an_c, hn.at[pl.ds(rj, CR)])
            for z in range(CR // ZR):
                pltpu.sync_copy(zrow, aggp.at[pl.ds(rj + z * ZR, ZR)])
                pltpu.sync_copy(zrow, aggn.at[pl.ds(rj + z * ZR, ZR)])
            pltpu.sync_copy(h_c, out.at[cid, pl.ds(rj, CR)])
        plsc.subcore_barrier()
        return carry

    lax.fori_loop(0, PROP_STEP, step, 0)


_sc_call = pl.kernel(
    _sc_body,
    out_type=jax.ShapeDtypeStruct((2, NP, F), jnp.float32),
    mesh=plsc.VectorSubcoreMesh(core_axis_name="c", subcore_axis_name="s",
                                num_cores=2, num_subcores=NSUB),
    compiler_params=pltpu.CompilerParams(use_tc_tiling_on_sc=False,
                                         needs_layout_passes=False),
    scratch_types=[
        pltpu.VMEM_SHARED((NP, F), jnp.float32),
        pltpu.VMEM_SHARED((NP, F), jnp.float32),
        pltpu.VMEM_SHARED((NP, F), jnp.float32),
        pltpu.VMEM_SHARED((NP, F), jnp.float32),
        pltpu.VMEM_SHARED((NP,), jnp.float32),
        pltpu.VMEM_SHARED((NP,), jnp.float32),
        pltpu.VMEM_SHARED((NP,), jnp.float32),
        pltpu.VMEM_SHARED((NP,), jnp.float32),
        pltpu.VMEM((RB, IB), jnp.int32),
        pltpu.VMEM((RB, IB), jnp.int32),
        pltpu.VMEM((RB, IB), jnp.int32),
        pltpu.VMEM((RB, IB), jnp.int32),
        *([pltpu.VMEM((IB, F), jnp.float32)] * NBUF),
        pltpu.VMEM((IB,), jnp.float32),
        pltpu.VMEM((ZR, F), jnp.float32),
        pltpu.VMEM((CR, F), jnp.float32),
        pltpu.VMEM((CR, F), jnp.float32),
        pltpu.VMEM((CR, F), jnp.float32),
        pltpu.VMEM((CR, F), jnp.float32),
        pltpu.VMEM((RPN, F), jnp.float32),
        pltpu.VMEM((RPN, F), jnp.float32),
        pltpu.VMEM((RPN, F), jnp.float32),
        pltpu.VMEM((RPN, F), jnp.float32),
        pltpu.VMEM((RPN,), jnp.float32),
        *([pltpu.SemaphoreType.DMA] * (2 * NBUF + 2)),
    ],
)


def kernel(x, edge_index, neg_edge_index, W1, b1, gamma, beta, W2, b2):
    scale = (gamma * np.float32(1.0 / np.sqrt(1.0 + 1e-5))).reshape(1, H_FEATS)
    shift = (b1 * scale[0] + beta).reshape(1, H_FEATS)
    ori = pl.pallas_call(
        _mlp_body,
        out_shape=jax.ShapeDtypeStruct((N, H_FEATS), jnp.float32),
    )(x, W1, scale, shift, W2, b2.reshape(1, H_FEATS))

    ori_split = jnp.pad(ori, ((0, NP - N), (0, 0))).reshape(NP, 2, F).transpose(1, 0, 2)

    padv = jnp.full((EP_PAD - E,), NP - 1, dtype=jnp.int32)
    src2d = jnp.concatenate(
        [edge_index[0], padv, neg_edge_index[0], padv]).reshape(ROWS_G, IB)
    dst2d = jnp.concatenate(
        [edge_index[1], padv, neg_edge_index[1], padv]).reshape(ROWS_G, IB)

    out = _sc_call(src2d, dst2d, ori_split)
    return out.transpose(1, 0, 2).reshape(NP, H_FEATS)[:N]


# speedup vs baseline: 1.3919x; 1.0091x over previous
import functools

import jax
import jax.numpy as jnp
import numpy as np
from jax import lax
from jax.experimental import pallas as pl
from jax.experimental.pallas import tpu as pltpu
from jax.experimental.pallas import tpu_sc as plsc

N = 10000
E = 320000
IN_FEATS = 128
H_FEATS = 32
PROP_STEP = 8

NP = 10240
F = 16
NSUB = 16
RPN = NP // NSUB
CR = 128
NSC = RPN // CR

IB = 128
EP_PAD = 327680
ROWS_G = 2 * EP_PAD // IB
ROWS_T = ROWS_G // NSUB
RB = 40
NBUF = 8
HALF = NBUF // 2
NCHUNK = ROWS_T // RB
DROWS_T = (EP_PAD // IB) // NSUB
DCHUNK = DROWS_T // RB
ZR = 32


def _rsqrt16(d):
    y = jnp.ones((16,), jnp.float32)
    for _ in range(7):
        y = jnp.where(d * y * y > 1.5, y * 0.5, y)
    for _ in range(5):
        y = y * (1.5 - 0.5 * d * y * y)
    return y


def _mlp_body(x_ref, w1_ref, a_ref, c_ref, w2_ref, b2_ref, o_ref):
    h = jnp.dot(x_ref[...], w1_ref[...], preferred_element_type=jnp.float32)
    h = jnp.maximum(h * a_ref[...] + c_ref[...], 0.0)
    o = jnp.dot(h, w2_ref[...], preferred_element_type=jnp.float32) + b2_ref[...]
    o_ref[0, 0:N, :] = o[:, 0:F]
    o_ref[1, 0:N, :] = o[:, F:H_FEATS]
    zpad = jnp.zeros((NP - N, F), jnp.float32)
    o_ref[0, N:NP, :] = zpad
    o_ref[1, N:NP, :] = zpad


def _sc_body(src2d, dst2d, ori, out,
             hp, hn, aggp, aggn, degop, degip, degon, degin,
             src_a, dst_a, src_b, dst_b,
             rb0, rb1, rb2, rb3, rb4, rb5, rb6, rb7, obuf, zrow,
             o_c, ap_c, an_c, h_c, sipb, sinb, sopb, sonb, stab,
             gs0, gs1, gs2, gs3, gs4, gs5, gs6, gs7,
             ss0, ss1, ss2, ss3, ss4, ss5, ss6, ss7, is0, is1):
    src_v, dst_v = src_a, dst_a
    rbufs = (rb0, rb1, rb2, rb3, rb4, rb5, rb6, rb7)
    gsems = (gs0, gs1, gs2, gs3, gs4, gs5, gs6, gs7)
    ssems = (ss0, ss1, ss2, ss3, ss4, ss5, ss6, ss7)
    cid = lax.axis_index("c")
    sid = lax.axis_index("s")
    r0 = sid * RPN
    ones = jnp.ones((16,), jnp.float32)
    zvec = jnp.zeros((16,), jnp.float32)

    def fill_zrow(i, _):
        zrow[i, :] = zvec
        return 0
    lax.fori_loop(0, ZR, fill_zrow, 0)

    def fill_ones(i, _):
        obuf[pl.ds(i * 16, 16)] = ones
        stab[pl.ds(i * 16, 16)] = zvec
        return 0
    lax.fori_loop(0, IB // 16, fill_ones, 0)

    def fill_stab(i, _):
        stab[pl.ds(IB + i * 16, 16)] = zvec
        return 0
    lax.fori_loop(0, (RPN - IB) // 16, fill_stab, 0)

    for tbl in (aggp, aggn):
        for j in range(RPN // ZR):
            pltpu.sync_copy(zrow, tbl.at[pl.ds(r0 + j * ZR, ZR)])
    for tbl in (degop, degip, degon, degin):
        pltpu.sync_copy(stab, tbl.at[pl.ds(r0, RPN)])
    plsc.subcore_barrier()

    for tbl, arr2d, base in ((degop, src2d, 0), (degip, dst2d, 0),
                             (degon, src2d, EP_PAD // IB),
                             (degin, dst2d, EP_PAD // IB)):
        def dchunk(c, _, tbl=tbl, arr2d=arr2d, base=base):
            row0 = base + sid * DROWS_T + c * RB
            pltpu.sync_copy(arr2d.at[pl.ds(row0, RB)], src_v)

            sdescs = [None] * NBUF
            for r in range(RB):
                b = r % NBUF
                if sdescs[b] is not None:
                    sdescs[b].wait()
                sdescs[b] = pltpu.async_copy(obuf, tbl.at[src_v.at[r]],
                                             ssems[b], add=True)
            for b in range(NBUF):
                if sdescs[b] is not None:
                    sdescs[b].wait()
            return 0
        lax.fori_loop(0, DCHUNK, dchunk, 0)
    plsc.subcore_barrier()

    for tbl, sb in ((degop, sopb), (degip, sipb), (degon, sonb), (degin, sinb)):
        pltpu.sync_copy(tbl.at[pl.ds(r0, RPN)], stab)

        def srow(i, _):
            d = stab[pl.ds(i * 16, 16)]
            stab[pl.ds(i * 16, 16)] = _rsqrt16(jnp.maximum(d, 1.0))
            return 0
        lax.fori_loop(0, RPN // 16, srow, 0)

        def sexp(i, _, sb=sb):
            sb[i, :] = plsc.load_gather(stab, [jnp.full((16,), i, jnp.int32)])
            return 0
        lax.fori_loop(0, RPN, sexp, 0)

    for j in range(NSC):
        rj = r0 + j * CR
        pltpu.sync_copy(ori.at[cid, pl.ds(rj, CR)], o_c)

        def irow(i, _, j=j):
            hrow = o_c[i, :]
            ap_c[i, :] = hrow * sopb[j * CR + i, :]
            an_c[i, :] = hrow * sonb[j * CR + i, :]
            return 0
        lax.fori_loop(0, CR, irow, 0)
        pltpu.sync_copy(ap_c, hp.at[pl.ds(rj, CR)])
        pltpu.sync_copy(an_c, hn.at[pl.ds(rj, CR)])
    plsc.subcore_barrier()

    def step(_, carry):
        def ring(sv, dv, htab, atab):
            gdescs = [None] * NBUF
            sdescs = [None] * NBUF
            for b in range(HALF):
                gdescs[b] = pltpu.async_copy(htab.at[sv.at[b]], rbufs[b],
                                             gsems[b])
            for r in range(RB):
                b = r % NBUF
                gdescs[b].wait()
                sdescs[b] = pltpu.async_copy(rbufs[b], atab.at[dv.at[r]],
                                             ssems[b], add=True)
                rg = r + HALF
                if rg < RB:
                    b2 = rg % NBUF
                    if sdescs[b2] is not None:
                        sdescs[b2].wait()
                        sdescs[b2] = None
                    gdescs[b2] = pltpu.async_copy(htab.at[sv.at[rg]],
                                                  rbufs[b2], gsems[b2])
            for b in range(NBUF):
                if sdescs[b] is not None:
                    sdescs[b].wait()

        def edge_chunk(c, __, htab=None, atab=None):
            row1 = sid * ROWS_T + (c + 1) * RB

            def one_parity(cs, cd, ns, nd):
                @pl.when(c < NCHUNK - 1)
                def _():
                    pltpu.async_copy(src2d.at[pl.ds(row1, RB)], ns, is0)
                    pltpu.async_copy(dst2d.at[pl.ds(row1, RB)], nd, is1)
                ring(cs, cd, htab, atab)
                @pl.when(c < NCHUNK - 1)
                def _():
                    pltpu.make_async_copy(src2d.at[pl.ds(row1, RB)], ns,
                                          is0).wait()
                    pltpu.make_async_copy(dst2d.at[pl.ds(row1, RB)], nd,
                                          is1).wait()

            @pl.when(c % 2 == 0)
            def _():
                one_parity(src_a, dst_a, src_b, dst_b)

            @pl.when(c % 2 == 1)
            def _():
                one_parity(src_b, dst_b, src_a, dst_a)
            return 0

        row00 = sid * ROWS_T
        pltpu.sync_copy(src2d.at[pl.ds(row00, RB)], src_a)
        pltpu.sync_copy(dst2d.at[pl.ds(row00, RB)], dst_a)

        @pl.when(sid < 8)
        def _():
            lax.fori_loop(0, NCHUNK, functools.partial(edge_chunk, htab=hp, atab=aggp), 0)

        @pl.when(sid >= 8)
        def _():
            lax.fori_loop(0, NCHUNK, functools.partial(edge_chunk, htab=hn, atab=aggn), 0)

        plsc.subcore_barrier()

        for j in range(NSC):
            rj = r0 + j * CR
            pltpu.sync_copy(aggp.at[pl.ds(rj, CR)], ap_c)
            pltpu.sync_copy(aggn.at[pl.ds(rj, CR)], an_c)
            pltpu.sync_copy(ori.at[cid, pl.ds(rj, CR)], o_c)

            def prow(i, __, j=j):
                hrow = (ap_c[i, :] * sipb[j * CR + i, :]
                        - an_c[i, :] * sinb[j * CR + i, :] + o_c[i, :])
                h_c[i, :] = hrow
                ap_c[i, :] = hrow * sopb[j * CR + i, :]
                an_c[i, :] = hrow * sonb[j * CR + i, :]
                return 0
            lax.fori_loop(0, CR, prow, 0)
            pltpu.sync_copy(ap_c, hp.at[pl.ds(rj, CR)])
            pltpu.sync_copy(an_c, hn.at[pl.ds(rj, CR)])
            for z in range(CR // ZR):
                pltpu.sync_copy(zrow, aggp.at[pl.ds(rj + z * ZR, ZR)])
                pltpu.sync_copy(zrow, aggn.at[pl.ds(rj + z * ZR, ZR)])
            pltpu.sync_copy(h_c, out.at[cid, pl.ds(rj, CR)])
        plsc.subcore_barrier()
        return carry

    lax.fori_loop(0, PROP_STEP, step, 0)


_sc_call = pl.kernel(
    _sc_body,
    out_type=jax.ShapeDtypeStruct((2, NP, F), jnp.float32),
    mesh=plsc.VectorSubcoreMesh(core_axis_name="c", subcore_axis_name="s",
                                num_cores=2, num_subcores=NSUB),
    compiler_params=pltpu.CompilerParams(use_tc_tiling_on_sc=False,
                                         needs_layout_passes=False),
    scratch_types=[
        pltpu.VMEM_SHARED((NP, F), jnp.float32),
        pltpu.VMEM_SHARED((NP, F), jnp.float32),
        pltpu.VMEM_SHARED((NP, F), jnp.float32),
        pltpu.VMEM_SHARED((NP, F), jnp.float32),
        pltpu.VMEM_SHARED((NP,), jnp.float32),
        pltpu.VMEM_SHARED((NP,), jnp.float32),
        pltpu.VMEM_SHARED((NP,), jnp.float32),
        pltpu.VMEM_SHARED((NP,), jnp.float32),
        pltpu.VMEM((RB, IB), jnp.int32),
        pltpu.VMEM((RB, IB), jnp.int32),
        pltpu.VMEM((RB, IB), jnp.int32),
        pltpu.VMEM((RB, IB), jnp.int32),
        *([pltpu.VMEM((IB, F), jnp.float32)] * NBUF),
        pltpu.VMEM((IB,), jnp.float32),
        pltpu.VMEM((ZR, F), jnp.float32),
        pltpu.VMEM((CR, F), jnp.float32),
        pltpu.VMEM((CR, F), jnp.float32),
        pltpu.VMEM((CR, F), jnp.float32),
        pltpu.VMEM((CR, F), jnp.float32),
        pltpu.VMEM((RPN, F), jnp.float32),
        pltpu.VMEM((RPN, F), jnp.float32),
        pltpu.VMEM((RPN, F), jnp.float32),
        pltpu.VMEM((RPN, F), jnp.float32),
        pltpu.VMEM((RPN,), jnp.float32),
        *([pltpu.SemaphoreType.DMA] * (2 * NBUF + 2)),
    ],
)


def kernel(x, edge_index, neg_edge_index, W1, b1, gamma, beta, W2, b2):
    scale = (gamma * np.float32(1.0 / np.sqrt(1.0 + 1e-5))).reshape(1, H_FEATS)
    shift = (b1 * scale[0] + beta).reshape(1, H_FEATS)
    ori_split = pl.pallas_call(
        _mlp_body,
        out_shape=jax.ShapeDtypeStruct((2, NP, F), jnp.float32),
    )(x, W1, scale, shift, W2, b2.reshape(1, H_FEATS))

    padv = jnp.full((EP_PAD - E,), NP - 1, dtype=jnp.int32)
    src2d = jnp.concatenate(
        [edge_index[0], padv, neg_edge_index[0], padv]).reshape(ROWS_G, IB)
    dst2d = jnp.concatenate(
        [edge_index[1], padv, neg_edge_index[1], padv]).reshape(ROWS_G, IB)

    out = _sc_call(src2d, dst2d, ori_split)
    return out.transpose(1, 0, 2).reshape(NP, H_FEATS)[:N]
